# Initial kernel scaffold; baseline (speedup 1.0000x reference)
import jax
import jax.numpy as jnp
from jax.experimental import pallas as pl


def kernel(x, edge_index, batch, mask, dataRef_x, dataRef_edge_index, conv0_W, conv0_b, conv1_W, conv1_b, ref0_W, ref0_b, ref1_W, ref1_b, ref2_W, ref2_b, mlp_W1, mlp_W2, mlp_W3, mlp_b3):
    raise NotImplementedError("write your pallas kernel here")


# speedup vs baseline: 4.0028x; 4.0028x over previous
import functools
import math

import jax
import jax.numpy as jnp
import numpy as np
from jax import lax
from jax.experimental import pallas as pl
from jax.experimental.pallas import tpu as pltpu
from jax.experimental.pallas import tpu_sc as plsc

F32 = jnp.float32
I32 = jnp.int32

HID = 64
WIN = 10

N_BIG = 50000
NT_BIG = 50176
E_BIG = 800000
EPB_BIG64 = 98
EPB_BIG16 = 49

N_REF = 2191
NT_REF = 2304
E_REF = 35056
EPB_REF64 = 5
EPB_REF16 = 3

NSUB = 16
NCORE = 2
KROW = 4


@functools.lru_cache(maxsize=None)
def _make_agg(nt, f, epb, table_rows):
    nr_sc = NSUB * epb * KROW
    dr = nt // NSUB
    mesh = plsc.VectorSubcoreMesh(core_axis_name="c", subcore_axis_name="s",
                                  num_cores=NCORE, num_subcores=NSUB)

    def body(table, gidx, sidx, zeros, out, gv, sv, rows, acc, gsem, ssem):
        c = lax.axis_index("c")
        s = lax.axis_index("s")
        pltpu.sync_copy(zeros, acc.at[pl.ds(s * dr, dr)])
        plsc.subcore_barrier()

        def blk(b, carry):
            r0 = c * nr_sc + s * (epb * KROW) + b * KROW
            pltpu.sync_copy(gidx.at[pl.ds(r0, KROW)], gv)
            pltpu.sync_copy(sidx.at[pl.ds(r0, KROW)], sv)
            cps = [pltpu.async_copy(table.at[gv.at[k]], rows.at[k], gsem)
                   for k in range(KROW)]
            for cp in cps:
                cp.wait()
            cps = [pltpu.async_copy(rows.at[k], acc.at[sv.at[k]], ssem, add=True)
                   for k in range(KROW)]
            for cp in cps:
                cp.wait()
            return carry

        lax.fori_loop(0, epb, blk, 0)
        plsc.subcore_barrier()
        pltpu.sync_copy(acc.at[pl.ds(s * dr, dr)],
                        out.at[pl.ds(c * nt + s * dr, dr)])

    return pl.kernel(
        body,
        out_type=jax.ShapeDtypeStruct((2 * nt, f), F32),
        mesh=mesh,
        scratch_types=[
            pltpu.VMEM((KROW, 128), I32),
            pltpu.VMEM((KROW, 128), I32),
            pltpu.VMEM((KROW, 128, f), F32),
            pltpu.VMEM_SHARED((nt, f), F32),
            pltpu.SemaphoreType.DMA,
            pltpu.SemaphoreType.DMA,
        ],
        compiler_params=pltpu.CompilerParams(use_tc_tiling_on_sc=False),
        name=f"sc_agg_{nt}_{f}_{epb}",
    )


def _pad1(a, length, fill):
    return jnp.concatenate(
        [a, jnp.full((length - a.shape[0],), fill, a.dtype)])


def _fs_idx(row, col, nt, nr_sc):
    ep = nr_sc * 128
    rp = _pad1(row, ep, I32(0))
    cp = _pad1(col, ep, jnp.int32(nt - 1))
    g = jnp.concatenate([rp, rp + nt]).reshape(2 * nr_sc, 128)
    s = jnp.concatenate([cp, cp]).reshape(2 * nr_sc, 128)
    return g, s


def _es_idx(gsrc, ssrc, nt, nr_sc, gfill):
    e = gsrc.shape[0]
    h0 = e // 2
    cap = nr_sc * 128
    g = jnp.concatenate([_pad1(gsrc[:h0], cap, gfill),
                         _pad1(gsrc[h0:], cap, gfill)]).reshape(2 * nr_sc, 128)
    s = jnp.concatenate([_pad1(ssrc[:h0], cap, jnp.int32(nt - 1)),
                         _pad1(ssrc[h0:], cap, jnp.int32(nt - 1))]
                        ).reshape(2 * nr_sc, 128)
    return g, s


@functools.lru_cache(maxsize=None)
def _make_layer(nt, nreal, blk):
    grid = nt // blk

    def body(x_ref, w_ref, b_ref, z_ref, s_ref, q_ref):
        i = pl.program_id(0)
        z = jnp.tanh(jnp.dot(x_ref[...], w_ref[...],
                             preferred_element_type=F32) + b_ref[0:1, :])
        z_ref[...] = z
        r = i * blk + lax.broadcasted_iota(I32, (blk, HID), 0)
        zm = jnp.where(r < nreal, z, 0.0)

        @pl.when(i == 0)
        def _():
            s_ref[...] = jnp.zeros_like(s_ref)
            q_ref[...] = jnp.zeros_like(q_ref)

        s_ref[0:1, :] += jnp.sum(zm, axis=0, keepdims=True)
        q_ref[0:1, :] += jnp.sum(zm * zm, axis=0, keepdims=True)

    return pl.pallas_call(
        body,
        grid=(grid,),
        in_specs=[
            pl.BlockSpec((blk, 3 * HID), lambda i: (i, 0)),
            pl.BlockSpec((3 * HID, HID), lambda i: (0, 0)),
            pl.BlockSpec((8, HID), lambda i: (0, 0)),
        ],
        out_specs=[
            pl.BlockSpec((blk, HID), lambda i: (i, 0)),
            pl.BlockSpec((8, HID), lambda i: (0, 0)),
            pl.BlockSpec((8, HID), lambda i: (0, 0)),
        ],
        out_shape=[
            jax.ShapeDtypeStruct((nt, HID), F32),
            jax.ShapeDtypeStruct((8, HID), F32),
            jax.ShapeDtypeStruct((8, HID), F32),
        ],
        name=f"tc_layer_{nt}",
    )


@functools.lru_cache(maxsize=None)
def _make_pool(nt, blk):
    grid = nt // blk

    def body(m_ref, h_ref, p_ref):
        i = pl.program_id(0)

        @pl.when(i == 0)
        def _():
            p_ref[...] = jnp.zeros_like(p_ref)

        p_ref[...] += lax.dot_general(
            m_ref[...], h_ref[...], (((0,), (0,)), ((), ())),
            preferred_element_type=F32)

    return pl.pallas_call(
        body,
        grid=(grid,),
        in_specs=[
            pl.BlockSpec((blk, 32), lambda i: (i, 0)),
            pl.BlockSpec((blk, HID), lambda i: (i, 0)),
        ],
        out_specs=pl.BlockSpec((32, HID), lambda i: (0, 0)),
        out_shape=jax.ShapeDtypeStruct((32, HID), F32),
        name="tc_pool",
    )


@functools.lru_cache(maxsize=None)
def _make_vecmat(kdim, ndim):
    def body(x_ref, w_ref, o_ref):
        o_ref[...] = jnp.dot(x_ref[...], w_ref[...],
                             preferred_element_type=F32)

    return pl.pallas_call(
        body,
        in_specs=[pl.BlockSpec((8, kdim), lambda: (0, 0)),
                  pl.BlockSpec((kdim, ndim), lambda: (0, 0))],
        out_specs=pl.BlockSpec((8, ndim), lambda: (0, 0)),
        out_shape=jax.ShapeDtypeStruct((8, ndim), F32),
        name="tc_vecmat",
    )


@functools.lru_cache(maxsize=None)
def _make_mlp(nt, dmlp, blk):
    grid = nt // blk

    def body(x_ref, t_ref, w1_ref, w2_ref, w3_ref, b3_ref, o_ref):
        t1 = jnp.dot(x_ref[...], w1_ref[...],
                     preferred_element_type=F32) + t_ref[0:1, :]
        r1 = jnp.maximum(t1, 0.0)
        r2 = jnp.maximum(jnp.dot(r1, w2_ref[...],
                                 preferred_element_type=F32), 0.0)
        o_ref[...] = jnp.dot(r2, w3_ref[...],
                             preferred_element_type=F32) + b3_ref[0:1, :]

    return pl.pallas_call(
        body,
        grid=(grid,),
        in_specs=[
            pl.BlockSpec((blk, HID), lambda i: (i, 0)),
            pl.BlockSpec((8, dmlp), lambda i: (0, 0)),
            pl.BlockSpec((HID, dmlp), lambda i: (0, 0)),
            pl.BlockSpec((dmlp, dmlp), lambda i: (0, 0)),
            pl.BlockSpec((dmlp, 128), lambda i: (0, 0)),
            pl.BlockSpec((8, 128), lambda i: (0, 0)),
        ],
        out_specs=pl.BlockSpec((blk, 128), lambda i: (i, 0)),
        out_shape=jax.ShapeDtypeStruct((nt, 128), F32),
        name="tc_mlp",
    )


def _gnn(x, row, col, Ws, bs, n, nt, epb64, epb16, blk):
    agg16 = _make_agg(nt, 16, epb16, nt)
    agg64 = _make_agg(nt, 32, epb64, 2 * nt)
    nr16 = NSUB * epb16 * KROW
    nr64 = NSUB * epb64 * KROW
    zeros16 = jnp.zeros((nt // NSUB, 16), F32)
    zeros32 = jnp.zeros((nt // NSUB, 32), F32)
    ones_tab = jnp.ones((nt, 16), F32)

    gz, s_row = _es_idx(jnp.zeros_like(row), row, nt, nr16, I32(0))
    deg_o = agg16(ones_tab, gz, s_row, zeros16)
    deg = deg_o[:n, 0] + deg_o[nt:nt + n, 0]
    dis = jnp.where(deg > 0, lax.rsqrt(jnp.where(deg > 0, deg, 1.0)), 0.0)
    disp = _pad1(dis, nt, F32(0))

    g16, s16 = _es_idx(row, col, nt, nr16, jnp.int32(nt - 1))
    g64, s64 = _fs_idx(row, col, nt, nr64)

    fin = x.shape[1]
    h = None
    for li, (W, b) in enumerate(zip(Ws, bs)):
        if li == 0:
            g0 = jnp.pad(disp[:, None] * jnp.pad(
                x, ((0, nt - n), (0, 0))), ((0, 0), (0, 16 - fin)))
            a1o = agg16(g0, g16, s16, zeros16)
            a1 = (a1o[:nt] + a1o[nt:])[:, :fin]
            g1 = jnp.pad(-(disp * disp)[:, None] * a1, ((0, 0), (0, 16 - fin)))
            a2o = agg16(g1, g16, s16, zeros16)
            a2 = (a2o[:nt] + a2o[nt:])[:, :fin]
            xp = jnp.pad(x, ((0, nt - n), (0, HID - fin)))
            u1 = jnp.pad(disp[:, None] * a1, ((0, 0), (0, HID - fin)))
            u2 = jnp.pad(disp[:, None] * a2, ((0, 0), (0, HID - fin)))
            wpad = ((0, HID - fin), (0, 0))
            wc = jnp.concatenate([jnp.pad(W[0] - W[2], wpad),
                                  jnp.pad(-W[1], wpad),
                                  jnp.pad(-2.0 * W[2], wpad)], axis=0)
        else:
            g = disp[:, None] * h
            gpk = jnp.concatenate([g[:, :32], g[:, 32:]], axis=0)
            a1o = agg64(gpk, g64, s64, zeros32)
            a1 = jnp.concatenate([a1o[:nt], a1o[nt:]], axis=1)
            g1f = -(disp * disp)[:, None] * a1
            g1pk = jnp.concatenate([g1f[:, :32], g1f[:, 32:]], axis=0)
            a2o = agg64(g1pk, g64, s64, zeros32)
            a2 = jnp.concatenate([a2o[:nt], a2o[nt:]], axis=1)
            xp = h
            u1 = disp[:, None] * a1
            u2 = disp[:, None] * a2
            wc = jnp.concatenate([W[0] - W[2], -W[1], -2.0 * W[2]], axis=0)

        xcat = jnp.concatenate([xp, u1, u2], axis=1)
        bp = jnp.zeros((8, HID), F32).at[0].set(b)
        z, ssum, sq = _make_layer(nt, n, blk)(xcat, wc, bp)
        m = ssum[0] / n
        var = sq[0] / n - m * m
        h = (z - m) * lax.rsqrt(var + 1e-5)
    return h, dis


def _pool_matrix(mask, batch0, n, nt):
    maskb = mask.astype(bool)
    mi = maskb.astype(I32)
    r1 = jnp.cumsum(mi) - 1
    r0 = jnp.cumsum(1 - mi) - 1
    rank = jnp.where(maskb, r1, r0)
    n_main = (n - mi.sum()).astype(I32) + batch0
    n_back = jnp.int32(n) - n_main
    cols = []
    for grp, cnt in ((0, n_main), (1, n_back)):
        inb = maskb == (grp == 1)
        for w in range(WIN):
            s = (w * cnt) // WIN
            e = -((-(w + 1) * cnt) // WIN)
            memb = (rank >= s) & (rank < e) & inb
            cols.append(memb.astype(F32) / (e - s).astype(F32))
    m = jnp.stack(cols, axis=1)
    return jnp.pad(m, ((0, nt - n), (0, 32 - 2 * WIN)))


def kernel(x, edge_index, batch, mask, dataRef_x, dataRef_edge_index,
           conv0_W, conv0_b, conv1_W, conv1_b,
           ref0_W, ref0_b, ref1_W, ref1_b, ref2_W, ref2_b,
           mlp_W1, mlp_W2, mlp_W3, mlp_b3):
    row, col = edge_index[0], edge_index[1]
    h2, _ = _gnn(x, row, col, (conv0_W, conv1_W), (conv0_b, conv1_b),
                 N_BIG, NT_BIG, EPB_BIG64, EPB_BIG16, 256)

    mp = _pool_matrix(mask, batch[0], N_BIG, NT_BIG)
    p = _make_pool(NT_BIG, 256)(mp, h2)
    x_main = p[:WIN][None]
    x_back = p[WIN:2 * WIN][None]
    full = jnp.concatenate([p[:WIN].reshape(-1), p[WIN:2 * WIN].reshape(-1)])

    rrow, rcol = dataRef_edge_index[0], dataRef_edge_index[1]
    xr, _ = _gnn(dataRef_x, rrow, rcol, (ref0_W, ref1_W, ref2_W),
                 (ref0_b, ref1_b, ref2_b),
                 N_REF, NT_REF, EPB_REF64, EPB_REF16, 128)

    dmlp = mlp_W1.shape[0]
    cbn = F32(1.0 / math.sqrt(1.0 + 1e-5))
    w1s = mlp_W1 * cbn
    fullp = jnp.zeros((8, dmlp - HID), F32).at[0].set(full)
    tsh = _make_vecmat(dmlp - HID, dmlp)(fullp, w1s[HID:])
    w3p = jnp.pad(mlp_W3, ((0, 0), (0, 128 - mlp_W3.shape[1])))
    b3p = jnp.zeros((8, 128), F32).at[0, :mlp_W3.shape[1]].set(mlp_b3)
    out = _make_mlp(NT_REF, dmlp, 128)(xr, tsh, w1s[:HID], mlp_W2 * cbn,
                                       w3p, b3p)
    res = out[:N_REF, :mlp_W3.shape[1]]
    return (res, x_back, x_main)


# speedup vs baseline: 9.3104x; 2.3260x over previous
import functools
import math

import jax
import jax.numpy as jnp
import numpy as np
from jax import lax
from jax.experimental import pallas as pl
from jax.experimental.pallas import tpu as pltpu
from jax.experimental.pallas import tpu_sc as plsc

F32 = jnp.float32
I32 = jnp.int32

HID = 64
WIN = 10

N_BIG = 50000
NT_BIG = 50176
E_BIG = 800000
EPB_BIG64 = 98
EPB_BIG16 = 49

N_REF = 2191
NT_REF = 2304
E_REF = 35056
EPB_REF64 = 5
EPB_REF16 = 3

NSUB = 16
NCORE = 2
KROW = 4


@functools.lru_cache(maxsize=None)
def _make_agg(nt, f, epb, table_rows):
    nr_sc = NSUB * epb * KROW
    dr = nt // NSUB
    mesh = plsc.VectorSubcoreMesh(core_axis_name="c", subcore_axis_name="s",
                                  num_cores=NCORE, num_subcores=NSUB)

    def body(table, gidx, sidx, zeros, out, gv, sv, rows, acc, gsem, ssem):
        c = lax.axis_index("c")
        s = lax.axis_index("s")
        pltpu.sync_copy(zeros, acc.at[pl.ds(s * dr, dr)])
        plsc.subcore_barrier()

        def blk(b, carry):
            r0 = c * nr_sc + s * (epb * KROW) + b * KROW
            pltpu.sync_copy(gidx.at[pl.ds(r0, KROW)], gv)
            pltpu.sync_copy(sidx.at[pl.ds(r0, KROW)], sv)
            cps = [pltpu.async_copy(table.at[gv.at[k]], rows.at[k], gsem)
                   for k in range(KROW)]
            for cp in cps:
                cp.wait()
            cps = [pltpu.async_copy(rows.at[k], acc.at[sv.at[k]], ssem, add=True)
                   for k in range(KROW)]
            for cp in cps:
                cp.wait()
            return carry

        lax.fori_loop(0, epb, blk, 0)
        plsc.subcore_barrier()
        pltpu.sync_copy(acc.at[pl.ds(s * dr, dr)],
                        out.at[pl.ds(c * nt + s * dr, dr)])

    return pl.kernel(
        body,
        out_type=jax.ShapeDtypeStruct((2 * nt, f), F32),
        mesh=mesh,
        scratch_types=[
            pltpu.VMEM((KROW, 128), I32),
            pltpu.VMEM((KROW, 128), I32),
            pltpu.VMEM((KROW, 128, f), F32),
            pltpu.VMEM_SHARED((nt, f), F32),
            pltpu.SemaphoreType.DMA,
            pltpu.SemaphoreType.DMA,
        ],
        compiler_params=pltpu.CompilerParams(use_tc_tiling_on_sc=False),
        name=f"sc_agg_{nt}_{f}_{epb}",
    )


def _pad1(a, length, fill):
    return jnp.concatenate(
        [a, jnp.full((length - a.shape[0],), fill, a.dtype)])


def _fs_idx(row, col, nt, nr_sc):
    ep = nr_sc * 128
    rp = _pad1(row, ep, I32(0))
    cp = _pad1(col, ep, jnp.int32(nt - 1))
    g = jnp.concatenate([rp, rp + nt]).reshape(2 * nr_sc, 128)
    s = jnp.concatenate([cp, cp]).reshape(2 * nr_sc, 128)
    return g, s


def _es_idx(gsrc, ssrc, nt, nr_sc, gfill):
    e = gsrc.shape[0]
    h0 = e // 2
    cap = nr_sc * 128
    g = jnp.concatenate([_pad1(gsrc[:h0], cap, gfill),
                         _pad1(gsrc[h0:], cap, gfill)]).reshape(2 * nr_sc, 128)
    s = jnp.concatenate([_pad1(ssrc[:h0], cap, jnp.int32(nt - 1)),
                         _pad1(ssrc[h0:], cap, jnp.int32(nt - 1))]
                        ).reshape(2 * nr_sc, 128)
    return g, s


@functools.lru_cache(maxsize=None)
def _make_layer(nt, nreal, blk):
    grid = nt // blk

    def body(x_ref, w_ref, b_ref, z_ref, s_ref, q_ref):
        i = pl.program_id(0)
        z = jnp.tanh(jnp.dot(x_ref[...], w_ref[...],
                             preferred_element_type=F32) + b_ref[0:1, :])
        z_ref[...] = z
        r = i * blk + lax.broadcasted_iota(I32, (blk, HID), 0)
        zm = jnp.where(r < nreal, z, 0.0)

        @pl.when(i == 0)
        def _():
            s_ref[...] = jnp.zeros_like(s_ref)
            q_ref[...] = jnp.zeros_like(q_ref)

        s_ref[0:1, :] += jnp.sum(zm, axis=0, keepdims=True)
        q_ref[0:1, :] += jnp.sum(zm * zm, axis=0, keepdims=True)

    return pl.pallas_call(
        body,
        grid=(grid,),
        in_specs=[
            pl.BlockSpec((blk, 3 * HID), lambda i: (i, 0)),
            pl.BlockSpec((3 * HID, HID), lambda i: (0, 0)),
            pl.BlockSpec((8, HID), lambda i: (0, 0)),
        ],
        out_specs=[
            pl.BlockSpec((blk, HID), lambda i: (i, 0)),
            pl.BlockSpec((8, HID), lambda i: (0, 0)),
            pl.BlockSpec((8, HID), lambda i: (0, 0)),
        ],
        out_shape=[
            jax.ShapeDtypeStruct((nt, HID), F32),
            jax.ShapeDtypeStruct((8, HID), F32),
            jax.ShapeDtypeStruct((8, HID), F32),
        ],
        name=f"tc_layer_{nt}",
    )


@functools.lru_cache(maxsize=None)
def _make_pool(nt, blk):
    grid = nt // blk

    def body(m_ref, h_ref, p_ref):
        i = pl.program_id(0)

        @pl.when(i == 0)
        def _():
            p_ref[...] = jnp.zeros_like(p_ref)

        p_ref[...] += lax.dot_general(
            m_ref[...], h_ref[...], (((0,), (0,)), ((), ())),
            preferred_element_type=F32)

    return pl.pallas_call(
        body,
        grid=(grid,),
        in_specs=[
            pl.BlockSpec((blk, 32), lambda i: (i, 0)),
            pl.BlockSpec((blk, HID), lambda i: (i, 0)),
        ],
        out_specs=pl.BlockSpec((32, HID), lambda i: (0, 0)),
        out_shape=jax.ShapeDtypeStruct((32, HID), F32),
        name="tc_pool",
    )


@functools.lru_cache(maxsize=None)
def _make_vecmat(kdim, ndim):
    def body(x_ref, w_ref, o_ref):
        o_ref[...] = jnp.dot(x_ref[...], w_ref[...],
                             preferred_element_type=F32)

    return pl.pallas_call(
        body,
        in_specs=[pl.BlockSpec((8, kdim), lambda: (0, 0)),
                  pl.BlockSpec((kdim, ndim), lambda: (0, 0))],
        out_specs=pl.BlockSpec((8, ndim), lambda: (0, 0)),
        out_shape=jax.ShapeDtypeStruct((8, ndim), F32),
        name="tc_vecmat",
    )


@functools.lru_cache(maxsize=None)
def _make_mlp(nt, dmlp, blk):
    grid = nt // blk

    def body(x_ref, t_ref, w1_ref, w2_ref, w3_ref, b3_ref, o_ref):
        t1 = jnp.dot(x_ref[...], w1_ref[...],
                     preferred_element_type=F32) + t_ref[0:1, :]
        r1 = jnp.maximum(t1, 0.0)
        r2 = jnp.maximum(jnp.dot(r1, w2_ref[...],
                                 preferred_element_type=F32), 0.0)
        o_ref[...] = jnp.dot(r2, w3_ref[...],
                             preferred_element_type=F32) + b3_ref[0:1, :]

    return pl.pallas_call(
        body,
        grid=(grid,),
        in_specs=[
            pl.BlockSpec((blk, HID), lambda i: (i, 0)),
            pl.BlockSpec((8, dmlp), lambda i: (0, 0)),
            pl.BlockSpec((HID, dmlp), lambda i: (0, 0)),
            pl.BlockSpec((dmlp, dmlp), lambda i: (0, 0)),
            pl.BlockSpec((dmlp, 128), lambda i: (0, 0)),
            pl.BlockSpec((8, 128), lambda i: (0, 0)),
        ],
        out_specs=pl.BlockSpec((blk, 128), lambda i: (i, 0)),
        out_shape=jax.ShapeDtypeStruct((nt, 128), F32),
        name="tc_mlp",
    )


def _gnn(x, row, col, Ws, bs, n, nt, epb64, epb16, blk):
    agg16 = _make_agg(nt, 16, epb16, nt)
    agg64 = _make_agg(nt, 32, epb64, 2 * nt)
    nr16 = NSUB * epb16 * KROW
    nr64 = NSUB * epb64 * KROW
    zeros16 = jnp.zeros((nt // NSUB, 16), F32)
    zeros32 = jnp.zeros((nt // NSUB, 32), F32)
    ones_tab = jnp.ones((nt, 16), F32)

    gz, s_row = _es_idx(row, row, nt, nr16, jnp.int32(nt - 1))
    deg_o = agg16(ones_tab, gz, s_row, zeros16)
    deg = deg_o[:n, 0] + deg_o[nt:nt + n, 0]
    dis = jnp.where(deg > 0, lax.rsqrt(jnp.where(deg > 0, deg, 1.0)), 0.0)
    disp = _pad1(dis, nt, F32(0))

    g16, s16 = _es_idx(row, col, nt, nr16, jnp.int32(nt - 1))
    g64, s64 = _fs_idx(row, col, nt, nr64)

    fin = x.shape[1]
    h = None
    for li, (W, b) in enumerate(zip(Ws, bs)):
        if li == 0:
            g0 = jnp.pad(disp[:, None] * jnp.pad(
                x, ((0, nt - n), (0, 0))), ((0, 0), (0, 16 - fin)))
            a1o = agg16(g0, g16, s16, zeros16)
            a1 = (a1o[:nt] + a1o[nt:])[:, :fin]
            g1 = jnp.pad(-(disp * disp)[:, None] * a1, ((0, 0), (0, 16 - fin)))
            a2o = agg16(g1, g16, s16, zeros16)
            a2 = (a2o[:nt] + a2o[nt:])[:, :fin]
            xp = jnp.pad(x, ((0, nt - n), (0, HID - fin)))
            u1 = jnp.pad(disp[:, None] * a1, ((0, 0), (0, HID - fin)))
            u2 = jnp.pad(disp[:, None] * a2, ((0, 0), (0, HID - fin)))
            wpad = ((0, HID - fin), (0, 0))
            wc = jnp.concatenate([jnp.pad(W[0] - W[2], wpad),
                                  jnp.pad(-W[1], wpad),
                                  jnp.pad(-2.0 * W[2], wpad)], axis=0)
        else:
            g = disp[:, None] * h
            gpk = jnp.concatenate([g[:, :32], g[:, 32:]], axis=0)
            a1o = agg64(gpk, g64, s64, zeros32)
            a1 = jnp.concatenate([a1o[:nt], a1o[nt:]], axis=1)
            g1f = -(disp * disp)[:, None] * a1
            g1pk = jnp.concatenate([g1f[:, :32], g1f[:, 32:]], axis=0)
            a2o = agg64(g1pk, g64, s64, zeros32)
            a2 = jnp.concatenate([a2o[:nt], a2o[nt:]], axis=1)
            xp = h
            u1 = disp[:, None] * a1
            u2 = disp[:, None] * a2
            wc = jnp.concatenate([W[0] - W[2], -W[1], -2.0 * W[2]], axis=0)

        xcat = jnp.concatenate([xp, u1, u2], axis=1)
        bp = jnp.zeros((8, HID), F32).at[0].set(b)
        z, ssum, sq = _make_layer(nt, n, blk)(xcat, wc, bp)
        m = ssum[0] / n
        var = sq[0] / n - m * m
        h = (z - m) * lax.rsqrt(var + 1e-5)
    return h, dis


def _pool_matrix(mask, batch0, n, nt):
    maskb = mask.astype(bool)
    mi = maskb.astype(I32)
    r1 = jnp.cumsum(mi) - 1
    r0 = jnp.cumsum(1 - mi) - 1
    rank = jnp.where(maskb, r1, r0)
    n_main = (n - mi.sum()).astype(I32) + batch0
    n_back = jnp.int32(n) - n_main
    cols = []
    for grp, cnt in ((0, n_main), (1, n_back)):
        inb = maskb == (grp == 1)
        for w in range(WIN):
            s = (w * cnt) // WIN
            e = -((-(w + 1) * cnt) // WIN)
            memb = (rank >= s) & (rank < e) & inb
            cols.append(memb.astype(F32) / (e - s).astype(F32))
    m = jnp.stack(cols, axis=1)
    return jnp.pad(m, ((0, nt - n), (0, 32 - 2 * WIN)))


def kernel(x, edge_index, batch, mask, dataRef_x, dataRef_edge_index,
           conv0_W, conv0_b, conv1_W, conv1_b,
           ref0_W, ref0_b, ref1_W, ref1_b, ref2_W, ref2_b,
           mlp_W1, mlp_W2, mlp_W3, mlp_b3):
    row, col = edge_index[0], edge_index[1]
    h2, _ = _gnn(x, row, col, (conv0_W, conv1_W), (conv0_b, conv1_b),
                 N_BIG, NT_BIG, EPB_BIG64, EPB_BIG16, 256)

    mp = _pool_matrix(mask, batch[0], N_BIG, NT_BIG)
    p = _make_pool(NT_BIG, 256)(mp, h2)
    x_main = p[:WIN][None]
    x_back = p[WIN:2 * WIN][None]
    full = jnp.concatenate([p[:WIN].reshape(-1), p[WIN:2 * WIN].reshape(-1)])

    rrow, rcol = dataRef_edge_index[0], dataRef_edge_index[1]
    xr, _ = _gnn(dataRef_x, rrow, rcol, (ref0_W, ref1_W, ref2_W),
                 (ref0_b, ref1_b, ref2_b),
                 N_REF, NT_REF, EPB_REF64, EPB_REF16, 128)

    dmlp = mlp_W1.shape[0]
    cbn = F32(1.0 / math.sqrt(1.0 + 1e-5))
    w1s = mlp_W1 * cbn
    fullp = jnp.zeros((8, dmlp - HID), F32).at[0].set(full)
    tsh = _make_vecmat(dmlp - HID, dmlp)(fullp, w1s[HID:])
    w3p = jnp.pad(mlp_W3, ((0, 0), (0, 128 - mlp_W3.shape[1])))
    b3p = jnp.zeros((8, 128), F32).at[0, :mlp_W3.shape[1]].set(mlp_b3)
    out = _make_mlp(NT_REF, dmlp, 128)(xr, tsh, w1s[:HID], mlp_W2 * cbn,
                                       w3p, b3p)
    res = out[:N_REF, :mlp_W3.shape[1]]
    return (res, x_back, x_main)
